# baseline (speedup 1.0000x reference)
import jax
import jax.numpy as jnp
from jax import lax
from jax.experimental import pallas as pl
from jax.experimental.pallas import tpu as pltpu
from jax.experimental.pallas import tpu_sc as plsc

_BATCH = 16384
_SEQ = 50
_DIM = 32
_CLS = 2

_NW = 32
_BPW = _BATCH // _NW
_GRP = 16
_NGRP = _BPW // _GRP
_ROWS = _GRP * _SEQ
_CHUNK = 100
_NCHUNK = _ROWS // _CHUNK


def _qnet_kernel(x_hbm, we_hbm, bb_hbm, table_hbm, out_hbm,
                 idx_v, rows_a, rows_b, we_v, bb_v, out_v, sem_a, sem_b):
    wid = lax.axis_index("s") * 2 + lax.axis_index("c")

    pltpu.sync_copy(x_hbm.at[wid], idx_v)
    pltpu.sync_copy(we_hbm, we_v)
    pltpu.sync_copy(bb_hbm, bb_v)

    w00 = we_v[0, 0:16]
    w01 = we_v[0, 16:32]
    w10 = we_v[1, 0:16]
    w11 = we_v[1, 16:32]
    bb0 = bb_v[0, :]
    bb1 = bb_v[1, :]
    lane15 = lax.iota(jnp.int32, 16) == 15

    def gather_copies(g, rows_v, sem):
        return [pltpu.async_copy(
            table_hbm.at[idx_v.at[g * _NCHUNK + j]],
            rows_v.at[pl.ds(j * _CHUNK, _CHUNK), :],
            sem) for j in range(_NCHUNK)]

    def issue(g, rows_v, sem):
        gather_copies(g, rows_v, sem)

    def drain(g, rows_v, sem):
        for cp in gather_copies(g, rows_v, sem):
            cp.wait()

    def compute(g, rows_v):
        def batch_body(bi, c2):
            r0 = bi * _SEQ
            acc0 = rows_v[r0, 0:16]
            acc1 = rows_v[r0, 16:32]
            for s in range(1, _SEQ):
                acc0 = acc0 + rows_v[r0 + s, 0:16]
                acc1 = acc1 + rows_v[r0 + s, 16:32]
            cs0 = jnp.cumsum(acc0 * w00 + acc1 * w01) + bb0
            cs1 = jnp.cumsum(acc0 * w10 + acc1 * w11) + bb1
            o = (g * _GRP + bi) * _CLS
            plsc.store_compressed(out_v.at[pl.ds(o, 16)], cs0, mask=lane15)
            plsc.store_compressed(out_v.at[pl.ds(o + 1, 16)], cs1, mask=lane15)
            return c2

        lax.fori_loop(0, _GRP, batch_body, 0)

    bufs = ((rows_a, sem_a), (rows_b, sem_b))

    issue(0, rows_a, sem_a)

    def pair_body(j, carry):
        g0 = j * 2
        for p in (0, 1):
            g = g0 + p
            rows_v, sem = bufs[p]
            nrows, nsem = bufs[1 - p]
            drain(g, rows_v, sem)
            issue(g + 1, nrows, nsem)
            compute(g, rows_v)
        return carry

    lax.fori_loop(0, _NGRP // 2 - 1, pair_body, 0)

    drain(_NGRP - 2, rows_a, sem_a)
    issue(_NGRP - 1, rows_b, sem_b)
    compute(_NGRP - 2, rows_a)
    drain(_NGRP - 1, rows_b, sem_b)
    compute(_NGRP - 1, rows_b)

    pltpu.sync_copy(out_v.at[pl.ds(0, _BPW * _CLS)],
                    out_hbm.at[pl.ds(wid * _BPW * _CLS, _BPW * _CLS)])


def kernel(x, table, W, b):
    xr = x.reshape(_NW, _NGRP * _NCHUNK, _CHUNK)
    we = W / float(_SEQ)
    bb = jnp.broadcast_to(b[:, None], (_CLS, 16))

    mesh = plsc.VectorSubcoreMesh(core_axis_name="c", subcore_axis_name="s")
    f = pl.kernel(
        _qnet_kernel,
        mesh=mesh,
        compiler_params=pltpu.CompilerParams(
            needs_layout_passes=False, use_tc_tiling_on_sc=False),
        out_type=jax.ShapeDtypeStruct((_BATCH * _CLS,), jnp.float32),
        scratch_types=[
            pltpu.VMEM((_NGRP * _NCHUNK, _CHUNK), jnp.int32),
            pltpu.VMEM((_ROWS, _DIM), jnp.float32),
            pltpu.VMEM((_ROWS, _DIM), jnp.float32),
            pltpu.VMEM((_CLS, _DIM), jnp.float32),
            pltpu.VMEM((_CLS, 16), jnp.float32),
            pltpu.VMEM((_BPW * _CLS + 16,), jnp.float32),
            pltpu.SemaphoreType.DMA,
            pltpu.SemaphoreType.DMA,
        ],
    )
    return f(xr, we, bb, table).reshape(_BATCH, _CLS)


# speedup vs baseline: 27.5445x; 27.5445x over previous
import jax
import jax.numpy as jnp
from jax import lax
from jax.experimental import pallas as pl
from jax.experimental.pallas import tpu as pltpu
from jax.experimental.pallas import tpu_sc as plsc

_BATCH = 16384
_SEQ = 50
_DIM = 32
_CLS = 2

_NW = 32
_BPW = _BATCH // _NW
_GRP = 16
_NGRP = _BPW // _GRP
_ROWS = _GRP * _SEQ
_CHUNK = 100
_NCHUNK = _ROWS // _CHUNK


def _qnet_kernel(x_hbm, we_hbm, bb_hbm, table_hbm, out_hbm,
                 idx_v, rows_a, rows_b, we_v, bb_v, out_v, sem_a, sem_b):
    wid = lax.axis_index("s") * 2 + lax.axis_index("c")

    pltpu.sync_copy(x_hbm.at[wid], idx_v)
    pltpu.sync_copy(we_hbm, we_v)
    pltpu.sync_copy(bb_hbm, bb_v)

    w00 = we_v[0, 0:16]
    w01 = we_v[0, 16:32]
    w10 = we_v[1, 0:16]
    w11 = we_v[1, 16:32]
    bb0 = bb_v[0, :]
    bb1 = bb_v[1, :]
    lane15 = lax.iota(jnp.int32, 16) == 15

    def issue(g, rows_v, sem):
        for j in range(_NCHUNK):
            pltpu.async_copy(
                table_hbm.at[idx_v.at[g * _NCHUNK + j]],
                rows_v.at[pl.ds(j * _CHUNK, _CHUNK), :],
                sem)

    def drain(g, rows_v, sem):
        for j in range(_NCHUNK):
            pltpu.make_async_copy(
                table_hbm.at[idx_v.at[g * _NCHUNK + j]],
                rows_v.at[pl.ds(j * _CHUNK, _CHUNK), :],
                sem).wait()

    def compute(g, rows_v):
        def batch_body(bi, c2):
            r0 = bi * _SEQ
            acc0 = rows_v[r0, 0:16]
            acc1 = rows_v[r0, 16:32]
            for s in range(1, _SEQ):
                acc0 = acc0 + rows_v[r0 + s, 0:16]
                acc1 = acc1 + rows_v[r0 + s, 16:32]
            cs0 = jnp.cumsum(acc0 * w00 + acc1 * w01) + bb0
            cs1 = jnp.cumsum(acc0 * w10 + acc1 * w11) + bb1
            o = (g * _GRP + bi) * _CLS
            plsc.store_compressed(out_v.at[pl.ds(o, 16)], cs0, mask=lane15)
            plsc.store_compressed(out_v.at[pl.ds(o + 1, 16)], cs1, mask=lane15)
            return c2

        lax.fori_loop(0, _GRP, batch_body, 0)

    bufs = ((rows_a, sem_a), (rows_b, sem_b))

    issue(0, rows_a, sem_a)

    def pair_body(j, carry):
        g0 = j * 2
        for p in (0, 1):
            g = g0 + p
            rows_v, sem = bufs[p]
            nrows, nsem = bufs[1 - p]
            drain(g, rows_v, sem)
            issue(g + 1, nrows, nsem)
            compute(g, rows_v)
        return carry

    lax.fori_loop(0, _NGRP // 2 - 1, pair_body, 0)

    drain(_NGRP - 2, rows_a, sem_a)
    issue(_NGRP - 1, rows_b, sem_b)
    compute(_NGRP - 2, rows_a)
    drain(_NGRP - 1, rows_b, sem_b)
    compute(_NGRP - 1, rows_b)

    pltpu.sync_copy(out_v.at[pl.ds(0, _BPW * _CLS)],
                    out_hbm.at[pl.ds(wid * _BPW * _CLS, _BPW * _CLS)])


def kernel(x, table, W, b):
    xr = x.reshape(_NW, _NGRP * _NCHUNK, _CHUNK)
    we = W / float(_SEQ)
    bb = jnp.broadcast_to(b[:, None], (_CLS, 16))

    mesh = plsc.VectorSubcoreMesh(core_axis_name="c", subcore_axis_name="s")
    f = pl.kernel(
        _qnet_kernel,
        mesh=mesh,
        compiler_params=pltpu.CompilerParams(
            needs_layout_passes=False, use_tc_tiling_on_sc=False),
        out_type=jax.ShapeDtypeStruct((_BATCH * _CLS,), jnp.float32),
        scratch_types=[
            pltpu.VMEM((_NGRP * _NCHUNK, _CHUNK), jnp.int32),
            pltpu.VMEM((_ROWS, _DIM), jnp.float32),
            pltpu.VMEM((_ROWS, _DIM), jnp.float32),
            pltpu.VMEM((_CLS, _DIM), jnp.float32),
            pltpu.VMEM((_CLS, 16), jnp.float32),
            pltpu.VMEM((_BPW * _CLS + 16,), jnp.float32),
            pltpu.SemaphoreType.DMA,
            pltpu.SemaphoreType.DMA,
        ],
    )
    return f(xr, we, bb, table).reshape(_BATCH, _CLS)
